# Initial kernel scaffold; baseline (speedup 1.0000x reference)
import jax
import jax.numpy as jnp
from jax.experimental import pallas as pl


def kernel(h, edge_index, coord, We1, be1, We2, be2, Wn1, bn1, Wn2, bn2, Wc1, bc1, Wc2):
    raise NotImplementedError("write your pallas kernel here")


# speedup vs baseline: 2.3511x; 2.3511x over previous
import functools

import jax
import jax.numpy as jnp
from jax import lax
from jax.experimental import pallas as pl
from jax.experimental.pallas import tpu as pltpu
from jax.experimental.pallas import tpu_sc as plsc

N = 10000
E = 320000
D = 128
H = 128
EPS = 1e-08

CW = 144
NC = 2
NS = 16
NW = NC * NS
EPW = E // NW
GB = 80
NBLK = EPW // GB
NP = 10240
RPT = NP // NS


def _silu(x):
    return x * jax.nn.sigmoid(x)


def _gather_body(hx, row, col, src_out, tgt_out,
                 ridx_v, cidx_v, srows_v, trows_v, sem1, sem2):
    c = lax.axis_index("c")
    s = lax.axis_index("s")
    wid = s * NC + c
    base = wid * EPW

    def body(i, _):
        off = base + i * GB
        pltpu.sync_copy(row.at[pl.ds(off, GB)], ridx_v)
        pltpu.sync_copy(col.at[pl.ds(off, GB)], cidx_v)
        cp1 = pltpu.async_copy(hx.at[ridx_v], srows_v, sem1)
        cp2 = pltpu.async_copy(hx.at[cidx_v], trows_v, sem2)
        cp1.wait()
        cp2.wait()
        pltpu.sync_copy(srows_v, src_out.at[pl.ds(off, GB)])
        pltpu.sync_copy(trows_v, tgt_out.at[pl.ds(off, GB)])
        return 0

    lax.fori_loop(0, NBLK, body, 0)


def _sc_gather(hx, row, col):
    mesh = plsc.VectorSubcoreMesh(core_axis_name="c", subcore_axis_name="s")
    k = pl.kernel(
        _gather_body,
        out_type=(jax.ShapeDtypeStruct((E, CW), jnp.float32),
                  jax.ShapeDtypeStruct((E, CW), jnp.float32)),
        mesh=mesh,
        scratch_types=[
            pltpu.VMEM((GB,), jnp.int32),
            pltpu.VMEM((GB,), jnp.int32),
            pltpu.VMEM((GB, CW), jnp.float32),
            pltpu.VMEM((GB, CW), jnp.float32),
            pltpu.SemaphoreType.DMA,
            pltpu.SemaphoreType.DMA,
        ],
        compiler_params=pltpu.CompilerParams(use_tc_tiling_on_sc=False),
    )
    return k(hx, row, col)


def _scatter_body(packed, ridx, zeros_hbm, out, idx_v, vals_v, acc):
    c = lax.axis_index("c")
    s = lax.axis_index("s")
    wid = s * NC + c
    base = wid * EPW

    pltpu.sync_copy(zeros_hbm, acc.at[pl.ds(s * RPT, RPT)])
    plsc.subcore_barrier()

    def body(i, _):
        off = base + i * GB
        pltpu.sync_copy(ridx.at[pl.ds(off, GB)], idx_v)
        pltpu.sync_copy(packed.at[pl.ds(off, GB)], vals_v)
        pltpu.sync_copy(vals_v, acc.at[idx_v], add=True)
        return 0

    lax.fori_loop(0, NBLK, body, 0)
    plsc.subcore_barrier()
    pltpu.sync_copy(acc.at[pl.ds(s * RPT, RPT)], out.at[c, pl.ds(s * RPT, RPT)])


def _sc_scatter(packed, ridx, zeros_hbm):
    mesh = plsc.VectorSubcoreMesh(core_axis_name="c", subcore_axis_name="s")
    k = pl.kernel(
        _scatter_body,
        out_type=jax.ShapeDtypeStruct((NC, NP, CW), jnp.float32),
        mesh=mesh,
        scratch_types=[
            pltpu.VMEM((GB,), jnp.int32),
            pltpu.VMEM((GB, CW), jnp.float32),
            pltpu.VMEM_SHARED((NP, CW), jnp.float32),
        ],
        compiler_params=pltpu.CompilerParams(use_tc_tiling_on_sc=False),
    )
    return k(packed, ridx, zeros_hbm)


BE = 2000


def _edge_block(src_ref, tgt_ref, we1s, we1t, we1r, be1, we2, be2,
                wc1, bc1, wc2, out_ref):
    s = src_ref[:, :D]
    t = tgt_ref[:, :D]
    cr = src_ref[:, D:CW]
    cc = tgt_ref[:, D:CW]
    cd = cr - cc
    radial = jnp.sum(cd * cd, axis=1, keepdims=True)
    norm = jnp.sqrt(radial) + EPS
    cdn = cd / norm
    pre1 = (jnp.dot(s, we1s[...], preferred_element_type=jnp.float32)
            + jnp.dot(t, we1t[...], preferred_element_type=jnp.float32)
            + radial * we1r[...] + be1[...])
    e1 = _silu(pre1)
    ef = _silu(jnp.dot(e1, we2[...], preferred_element_type=jnp.float32)
               + be2[...])
    c1 = _silu(jnp.dot(ef, wc1[...], preferred_element_type=jnp.float32)
               + bc1[...])
    cwt = jnp.sum(c1 * wc2[...], axis=1, keepdims=True)
    out_ref[:, :D] = ef
    out_ref[:, D:CW] = cdn * cwt


def _tc_edge(src_ext, tgt_ext, we1s, we1t, we1r, be1, we2, be2, wc1, bc1, wc2):
    nblk = E // BE
    full = lambda shape: pl.BlockSpec(shape, lambda i: (0,) * len(shape))
    return pl.pallas_call(
        _edge_block,
        grid=(nblk,),
        in_specs=[
            pl.BlockSpec((BE, CW), lambda i: (i, 0)),
            pl.BlockSpec((BE, CW), lambda i: (i, 0)),
            full((D, H)), full((D, H)), full((1, H)), full((1, H)),
            full((H, H)), full((1, H)),
            full((H, H)), full((1, H)), full((1, H)),
        ],
        out_specs=pl.BlockSpec((BE, CW), lambda i: (i, 0)),
        out_shape=jax.ShapeDtypeStruct((E, CW), jnp.float32),
    )(src_ext, tgt_ext, we1s, we1t, we1r, be1, we2, be2, wc1, bc1, wc2)


BN = 2000


def _node_block(h_ref, cp_ref, agg_ref, wn1h, wn1a, bn1, wn2, bn2,
                hout_ref, cout_ref):
    aggf = agg_ref[0] + agg_ref[1]
    agg = aggf[:, :D]
    h = h_ref[...]
    pre = (jnp.dot(h, wn1h[...], preferred_element_type=jnp.float32)
           + jnp.dot(agg, wn1a[...], preferred_element_type=jnp.float32)
           + bn1[...])
    hn = jnp.dot(_silu(pre), wn2[...], preferred_element_type=jnp.float32) \
        + bn2[...]
    hout_ref[...] = h + hn
    cout_ref[...] = cp_ref[...] + aggf[:, D:CW]


def _tc_node(h, coordp, aggp, wn1h, wn1a, bn1, wn2, bn2):
    nblk = N // BN
    full = lambda shape: pl.BlockSpec(shape, lambda i: (0,) * len(shape))
    return pl.pallas_call(
        _node_block,
        grid=(nblk,),
        in_specs=[
            pl.BlockSpec((BN, D), lambda i: (i, 0)),
            pl.BlockSpec((BN, 16), lambda i: (i, 0)),
            pl.BlockSpec((NC, BN, CW), lambda i: (0, i, 0)),
            full((D, H)), full((D, H)), full((1, H)),
            full((H, D)), full((1, D)),
        ],
        out_specs=[
            pl.BlockSpec((BN, D), lambda i: (i, 0)),
            pl.BlockSpec((BN, 16), lambda i: (i, 0)),
        ],
        out_shape=[
            jax.ShapeDtypeStruct((N, D), jnp.float32),
            jax.ShapeDtypeStruct((N, 16), jnp.float32),
        ],
    )(h, coordp, aggp, wn1h, wn1a, bn1, wn2, bn2)


def kernel(h, edge_index, coord, We1, be1, We2, be2, Wn1, bn1, Wn2, bn2,
           Wc1, bc1, Wc2):
    row = edge_index[0]
    col = edge_index[1]

    hx = jnp.concatenate(
        [h, coord, jnp.zeros((N, CW - D - 3), jnp.float32)], axis=1)

    src_ext, tgt_ext = _sc_gather(hx, row, col)

    packed = _tc_edge(
        src_ext, tgt_ext,
        We1[:, :D].T, We1[:, D:2 * D].T, We1[:, 2 * D:].T,
        be1.reshape(1, H), We2.T, be2.reshape(1, H),
        Wc1.T, bc1.reshape(1, H), Wc2.reshape(1, H))

    zeros_hbm = jnp.zeros((RPT, CW), jnp.float32)
    aggp = _sc_scatter(packed, row, zeros_hbm)

    coordp = jnp.pad(coord, ((0, 0), (0, 13)))
    h_out, coutp = _tc_node(
        h, coordp, aggp,
        Wn1[:, :D].T, Wn1[:, D:].T, bn1.reshape(1, H),
        Wn2.T, bn2.reshape(1, D))
    return (h_out, coutp[:, :3])


# speedup vs baseline: 2.3570x; 1.0025x over previous
import functools

import jax
import jax.numpy as jnp
from jax import lax
from jax.experimental import pallas as pl
from jax.experimental.pallas import tpu as pltpu
from jax.experimental.pallas import tpu_sc as plsc

N = 10000
E = 320000
D = 128
H = 128
EPS = 1e-08

CW = 144
NC = 2
NS = 16
NW = NC * NS
EPW = E // NW
GB = 80
NBLK = EPW // GB
NP = 10240
RPT = NP // NS


def _silu(x):
    return x * jax.nn.sigmoid(x)


def _gather_body(hx, row, col, src_out, tgt_out,
                 ridx_v, cidx_v, srows_v, trows_v, sem1, sem2):
    c = lax.axis_index("c")
    s = lax.axis_index("s")
    wid = s * NC + c
    base = wid * EPW

    def body(i, _):
        off = base + i * GB
        pltpu.sync_copy(row.at[pl.ds(off, GB)], ridx_v)
        pltpu.sync_copy(col.at[pl.ds(off, GB)], cidx_v)
        cp1 = pltpu.async_copy(hx.at[ridx_v], srows_v, sem1)
        cp2 = pltpu.async_copy(hx.at[cidx_v], trows_v, sem2)
        cp1.wait()
        cp2.wait()
        pltpu.sync_copy(srows_v, src_out.at[pl.ds(off, GB)])
        pltpu.sync_copy(trows_v, tgt_out.at[pl.ds(off, GB)])
        return 0

    lax.fori_loop(0, NBLK, body, 0)


def _sc_gather(hx, row, col):
    mesh = plsc.VectorSubcoreMesh(core_axis_name="c", subcore_axis_name="s")
    k = pl.kernel(
        _gather_body,
        out_type=(jax.ShapeDtypeStruct((E, CW), jnp.float32),
                  jax.ShapeDtypeStruct((E, CW), jnp.float32)),
        mesh=mesh,
        scratch_types=[
            pltpu.VMEM((GB,), jnp.int32),
            pltpu.VMEM((GB,), jnp.int32),
            pltpu.VMEM((GB, CW), jnp.float32),
            pltpu.VMEM((GB, CW), jnp.float32),
            pltpu.SemaphoreType.DMA,
            pltpu.SemaphoreType.DMA,
        ],
        compiler_params=pltpu.CompilerParams(use_tc_tiling_on_sc=False),
    )
    return k(hx, row, col)


def _scatter_body(packed, ridx, zeros_hbm, out, idx_v, vals_v, acc):
    c = lax.axis_index("c")
    s = lax.axis_index("s")
    wid = s * NC + c
    base = wid * EPW

    pltpu.sync_copy(zeros_hbm, acc.at[pl.ds(s * RPT, RPT)])
    plsc.subcore_barrier()

    def body(i, _):
        off = base + i * GB
        pltpu.sync_copy(ridx.at[pl.ds(off, GB)], idx_v)
        pltpu.sync_copy(packed.at[pl.ds(off, GB)], vals_v)
        pltpu.sync_copy(vals_v, acc.at[idx_v], add=True)
        return 0

    lax.fori_loop(0, NBLK, body, 0)
    plsc.subcore_barrier()
    pltpu.sync_copy(acc.at[pl.ds(s * RPT, RPT)], out.at[c, pl.ds(s * RPT, RPT)])


def _sc_scatter(packed, ridx, zeros_hbm):
    mesh = plsc.VectorSubcoreMesh(core_axis_name="c", subcore_axis_name="s")
    k = pl.kernel(
        _scatter_body,
        out_type=jax.ShapeDtypeStruct((NC, NP, CW), jnp.float32),
        mesh=mesh,
        scratch_types=[
            pltpu.VMEM((GB,), jnp.int32),
            pltpu.VMEM((GB, CW), jnp.float32),
            pltpu.VMEM_SHARED((NP, CW), jnp.float32),
        ],
        compiler_params=pltpu.CompilerParams(use_tc_tiling_on_sc=False),
    )
    return k(packed, ridx, zeros_hbm)


BE = 2000


def _edge_block(src_ref, tgt_ref, we1s, we1t, we1r, be1, we2, be2,
                wc1, bc1, wc2, out_ref):
    bf = jnp.bfloat16
    s = src_ref[:, :D].astype(bf)
    t = tgt_ref[:, :D].astype(bf)
    cr = src_ref[:, D:CW]
    cc = tgt_ref[:, D:CW]
    cd = cr - cc
    radial = jnp.sum(cd * cd, axis=1, keepdims=True)
    norm = jnp.sqrt(radial) + EPS
    cdn = cd / norm
    pre1 = (jnp.dot(s, we1s[...].astype(bf), preferred_element_type=jnp.float32)
            + jnp.dot(t, we1t[...].astype(bf), preferred_element_type=jnp.float32)
            + radial * we1r[...] + be1[...])
    e1 = _silu(pre1)
    ef = _silu(jnp.dot(e1.astype(bf), we2[...].astype(bf),
                       preferred_element_type=jnp.float32) + be2[...])
    c1 = _silu(jnp.dot(ef.astype(bf), wc1[...].astype(bf),
                       preferred_element_type=jnp.float32) + bc1[...])
    cwt = jnp.sum(c1 * wc2[...], axis=1, keepdims=True)
    out_ref[:, :D] = ef
    out_ref[:, D:CW] = cdn * cwt


def _tc_edge(src_ext, tgt_ext, we1s, we1t, we1r, be1, we2, be2, wc1, bc1, wc2):
    nblk = E // BE
    full = lambda shape: pl.BlockSpec(shape, lambda i: (0,) * len(shape))
    return pl.pallas_call(
        _edge_block,
        grid=(nblk,),
        in_specs=[
            pl.BlockSpec((BE, CW), lambda i: (i, 0)),
            pl.BlockSpec((BE, CW), lambda i: (i, 0)),
            full((D, H)), full((D, H)), full((1, H)), full((1, H)),
            full((H, H)), full((1, H)),
            full((H, H)), full((1, H)), full((1, H)),
        ],
        out_specs=pl.BlockSpec((BE, CW), lambda i: (i, 0)),
        out_shape=jax.ShapeDtypeStruct((E, CW), jnp.float32),
    )(src_ext, tgt_ext, we1s, we1t, we1r, be1, we2, be2, wc1, bc1, wc2)


BN = 2000


def _node_block(h_ref, cp_ref, agg_ref, wn1h, wn1a, bn1, wn2, bn2,
                hout_ref, cout_ref):
    bf = jnp.bfloat16
    aggf = agg_ref[0] + agg_ref[1]
    agg = aggf[:, :D]
    h = h_ref[...]
    pre = (jnp.dot(h.astype(bf), wn1h[...].astype(bf),
                   preferred_element_type=jnp.float32)
           + jnp.dot(agg.astype(bf), wn1a[...].astype(bf),
                     preferred_element_type=jnp.float32)
           + bn1[...])
    hn = jnp.dot(_silu(pre).astype(bf), wn2[...].astype(bf),
                 preferred_element_type=jnp.float32) + bn2[...]
    hout_ref[...] = h + hn
    cout_ref[...] = cp_ref[...] + aggf[:, D:CW]


def _tc_node(h, coordp, aggp, wn1h, wn1a, bn1, wn2, bn2):
    nblk = N // BN
    full = lambda shape: pl.BlockSpec(shape, lambda i: (0,) * len(shape))
    return pl.pallas_call(
        _node_block,
        grid=(nblk,),
        in_specs=[
            pl.BlockSpec((BN, D), lambda i: (i, 0)),
            pl.BlockSpec((BN, 16), lambda i: (i, 0)),
            pl.BlockSpec((NC, BN, CW), lambda i: (0, i, 0)),
            full((D, H)), full((D, H)), full((1, H)),
            full((H, D)), full((1, D)),
        ],
        out_specs=[
            pl.BlockSpec((BN, D), lambda i: (i, 0)),
            pl.BlockSpec((BN, 16), lambda i: (i, 0)),
        ],
        out_shape=[
            jax.ShapeDtypeStruct((N, D), jnp.float32),
            jax.ShapeDtypeStruct((N, 16), jnp.float32),
        ],
    )(h, coordp, aggp, wn1h, wn1a, bn1, wn2, bn2)


def kernel(h, edge_index, coord, We1, be1, We2, be2, Wn1, bn1, Wn2, bn2,
           Wc1, bc1, Wc2):
    row = edge_index[0]
    col = edge_index[1]

    hx = jnp.concatenate(
        [h, coord, jnp.zeros((N, CW - D - 3), jnp.float32)], axis=1)

    src_ext, tgt_ext = _sc_gather(hx, row, col)

    packed = _tc_edge(
        src_ext, tgt_ext,
        We1[:, :D].T, We1[:, D:2 * D].T, We1[:, 2 * D:].T,
        be1.reshape(1, H), We2.T, be2.reshape(1, H),
        Wc1.T, bc1.reshape(1, H), Wc2.reshape(1, H))

    zeros_hbm = jnp.zeros((RPT, CW), jnp.float32)
    aggp = _sc_scatter(packed, row, zeros_hbm)

    coordp = jnp.pad(coord, ((0, 0), (0, 13)))
    h_out, coutp = _tc_node(
        h, coordp, aggp,
        Wn1[:, :D].T, Wn1[:, D:].T, bn1.reshape(1, H),
        Wn2.T, bn2.reshape(1, D))
    return (h_out, coutp[:, :3])
